# baseline (speedup 1.0000x reference)
import jax
import jax.numpy as jnp
from jax.experimental import pallas as pl
from jax.experimental.pallas import tpu as pltpu

_B, _L, _N, _D, _F, _E, _K, _P, _LAYERS = 32, 96, 32, 16, 64, 4, 2, 96, 2
_EF = _E * _F
_BLK = 4
_T = _BLK * _L * _N
_TB = _L * _N
_CH = 1536


def _top2(lgT):
    f32 = jnp.float32

    def first_max_onehot(x):
        v = jnp.max(x, axis=0, keepdims=True)
        rows = []
        seen = jnp.zeros_like(v)
        for e in range(_E):
            eq = (x[e:e + 1] == v).astype(f32)
            rows.append(eq * (1.0 - seen))
            seen = jnp.maximum(seen, eq)
        return jnp.concatenate(rows, axis=0), v

    oh1, v1 = first_max_onehot(lgT)
    masked = jnp.where(oh1 > 0, -jnp.inf, lgT)
    oh2, v2 = first_max_onehot(masked)
    e2 = jnp.exp(v2 - v1)
    den = 1.0 + e2
    return oh1 * (1.0 / den) + oh2 * (e2 / den)


def _model_kernel(x_ref, startW_ref, startb_ref, wg_ref, W1_ref, b1_ref,
                  W2_ref, b2_ref, projW_ref, projb_ref,
                  dec_ref, bal_ref, imp_ref, load_ref, M_ref):
    b = pl.program_id(0)
    f32 = jnp.float32

    xt = x_ref[0]

    imp_parts = [[] for _ in range(_LAYERS)]
    load_parts = [[] for _ in range(_LAYERS)]
    for c in range(_T // _CH):
        outT = (startW_ref[:] * xt[:, c * _CH:(c + 1) * _CH]
                + startb_ref[:])
        for l in range(_LAYERS):
            lgT = jnp.dot(wg_ref[l], outT, preferred_element_type=f32)
            gatesT = _top2(lgT)
            imp_parts[l].append(jnp.sum(gatesT, axis=1, keepdims=True))
            load_parts[l].append(
                jnp.sum((gatesT > 0).astype(f32), axis=1, keepdims=True))

            hT = jax.nn.gelu(
                jnp.dot(W1_ref[l], outT, preferred_element_type=f32)
                + b1_ref[l])
            yT = None
            for e in range(_E):
                oeT = (jnp.dot(W2_ref[l][:, e * _F:(e + 1) * _F],
                               hT[e * _F:(e + 1) * _F],
                               preferred_element_type=f32)
                       + b2_ref[l][:, e:e + 1])
                term = gatesT[e:e + 1] * oeT
                yT = term if yT is None else yT + term
            outT = outT + yT

        for j in range(_CH // _N):
            off = c * _CH + j * _N
            k, r = off // _TB, (off % _TB) // _N
            M_ref[k, r * _D:(r + 1) * _D, :] = outT[:, j * _N:(j + 1) * _N]
    imps = [sum(p[1:], p[0]) for p in imp_parts]
    loads = [sum(p[1:], p[0]) for p in load_parts]
    for k in range(_BLK):
        dec_ref[k] = (jnp.dot(projW_ref[:], M_ref[k], preferred_element_type=f32)
                      + projb_ref[:])

    @pl.when(b == 0)
    def _init():
        for l in range(_LAYERS):
            imp_ref[l] = imps[l]
            load_ref[l] = loads[l]

    @pl.when(b != 0)
    def _acc():
        for l in range(_LAYERS):
            imp_ref[l] += imps[l]
            load_ref[l] += loads[l]

    @pl.when(b == _B // _BLK - 1)
    def _finalize():
        bal = jnp.zeros((1, 1), dtype=f32)
        for l in range(_LAYERS):
            for ref in (imp_ref, load_ref):
                v = ref[l]
                m = jnp.sum(v, keepdims=True) / _E
                var = jnp.sum((v - m) ** 2, keepdims=True) / (_E - 1)
                bal = bal + var / (m * m + 1e-10)
        bal_ref[:] = bal


def kernel(x_enc, x_mark_enc, x_dec, x_mark_dec, start_W, start_b, w_gate,
           W1, b1, W2, b2, proj_W, proj_b):
    f32 = jnp.float32
    wgT = jnp.transpose(w_gate, (0, 2, 1))
    W1T = jnp.transpose(W1, (0, 1, 3, 2)).reshape(_LAYERS, _EF, _D)
    b1T = b1.reshape(_LAYERS, _EF, 1)
    W2T = jnp.transpose(W2, (0, 3, 1, 2)).reshape(_LAYERS, _D, _EF)
    b2T = jnp.transpose(b2, (0, 2, 1))
    startWT = start_W.reshape(_D, 1)
    startbT = start_b.reshape(_D, 1)
    projWT = jnp.transpose(proj_W, (1, 0))
    projbT = proj_b.reshape(_P, 1)
    xp = x_enc.reshape(_B // _BLK, 1, _T)

    full = lambda shape: pl.BlockSpec(shape, lambda b: (0,) * len(shape))
    dec, bal = pl.pallas_call(
        _model_kernel,
        grid=(_B // _BLK,),
        in_specs=[
            pl.BlockSpec((1, 1, _T), lambda b: (b, 0, 0)),
            full((_D, 1)),
            full((_D, 1)),
            full((_LAYERS, _E, _D)),
            full((_LAYERS, _EF, _D)),
            full((_LAYERS, _EF, 1)),
            full((_LAYERS, _D, _EF)),
            full((_LAYERS, _D, _E)),
            full((_P, _L * _D)),
            full((_P, 1)),
        ],
        out_specs=[
            pl.BlockSpec((_BLK, _P, _N), lambda b: (b, 0, 0)),
            pl.BlockSpec((1, 1), lambda b: (0, 0)),
        ],
        out_shape=[
            jax.ShapeDtypeStruct((_B, _P, _N), f32),
            jax.ShapeDtypeStruct((1, 1), f32),
        ],
        scratch_shapes=[
            pltpu.VMEM((_LAYERS, _E, 1), f32),
            pltpu.VMEM((_LAYERS, _E, 1), f32),
            pltpu.VMEM((_BLK, _L * _D, _N), f32),
        ],
        compiler_params=pltpu.CompilerParams(
            dimension_semantics=("arbitrary",),
        ),
    )(xp, startWT, startbT, wgT, W1T, b1T, W2T, b2T, projWT, projbT)
    return dec, bal[0, 0]


# speedup vs baseline: 1.0873x; 1.0873x over previous
import jax
import jax.numpy as jnp
from jax.experimental import pallas as pl
from jax.experimental.pallas import tpu as pltpu

_B, _L, _N, _D, _F, _E, _K, _P, _LAYERS = 32, 96, 32, 16, 64, 4, 2, 96, 2
_EF = _E * _F
_BLK = 4
_T = _BLK * _L * _N
_TB = _L * _N
_CH = 3072


def _top2(lgT):
    f32 = jnp.float32

    def first_max_onehot(x):
        v = jnp.max(x, axis=0, keepdims=True)
        rows = []
        seen = jnp.zeros_like(v)
        for e in range(_E):
            eq = (x[e:e + 1] == v).astype(f32)
            rows.append(eq * (1.0 - seen))
            seen = jnp.maximum(seen, eq)
        return jnp.concatenate(rows, axis=0), v

    oh1, v1 = first_max_onehot(lgT)
    masked = jnp.where(oh1 > 0, -jnp.inf, lgT)
    oh2, v2 = first_max_onehot(masked)
    e2 = jnp.exp(v2 - v1)
    den = 1.0 + e2
    return oh1 * (1.0 / den) + oh2 * (e2 / den)


def _model_kernel(x_ref, startW_ref, startb_ref, wg_ref, W1_ref, b1_ref,
                  W2_ref, b2_ref, projW_ref, projb_ref,
                  dec_ref, bal_ref, imp_ref, load_ref, M_ref):
    b = pl.program_id(0)
    f32 = jnp.float32

    xt = x_ref[0]

    imp_parts = [[] for _ in range(_LAYERS)]
    load_parts = [[] for _ in range(_LAYERS)]
    for c in range(_T // _CH):
        outT = (startW_ref[:] * xt[:, c * _CH:(c + 1) * _CH]
                + startb_ref[:])
        for l in range(_LAYERS):
            lgT = jnp.dot(wg_ref[l], outT, preferred_element_type=f32)
            gatesT = _top2(lgT)
            imp_parts[l].append(jnp.sum(gatesT, axis=1, keepdims=True))
            load_parts[l].append(
                jnp.sum((gatesT > 0).astype(f32), axis=1, keepdims=True))

            hT = jax.nn.gelu(
                jnp.dot(W1_ref[l], outT, preferred_element_type=f32)
                + b1_ref[l])
            yT = None
            for e in range(_E):
                oeT = (jnp.dot(W2_ref[l][:, e * _F:(e + 1) * _F],
                               hT[e * _F:(e + 1) * _F],
                               preferred_element_type=f32)
                       + b2_ref[l][:, e:e + 1])
                term = gatesT[e:e + 1] * oeT
                yT = term if yT is None else yT + term
            outT = outT + yT

        for j in range(_CH // _N):
            off = c * _CH + j * _N
            k, r = off // _TB, (off % _TB) // _N
            M_ref[k, r * _D:(r + 1) * _D, :] = outT[:, j * _N:(j + 1) * _N]
    imps = [sum(p[1:], p[0]) for p in imp_parts]
    loads = [sum(p[1:], p[0]) for p in load_parts]
    for k in range(_BLK):
        dec_ref[k] = (jnp.dot(projW_ref[:], M_ref[k], preferred_element_type=f32)
                      + projb_ref[:])

    @pl.when(b == 0)
    def _init():
        for l in range(_LAYERS):
            imp_ref[l] = imps[l]
            load_ref[l] = loads[l]

    @pl.when(b != 0)
    def _acc():
        for l in range(_LAYERS):
            imp_ref[l] += imps[l]
            load_ref[l] += loads[l]

    @pl.when(b == _B // _BLK - 1)
    def _finalize():
        bal = jnp.zeros((1, 1), dtype=f32)
        for l in range(_LAYERS):
            for ref in (imp_ref, load_ref):
                v = ref[l]
                m = jnp.sum(v, keepdims=True) / _E
                var = jnp.sum((v - m) ** 2, keepdims=True) / (_E - 1)
                bal = bal + var / (m * m + 1e-10)
        bal_ref[:] = bal


def kernel(x_enc, x_mark_enc, x_dec, x_mark_dec, start_W, start_b, w_gate,
           W1, b1, W2, b2, proj_W, proj_b):
    f32 = jnp.float32
    wgT = jnp.transpose(w_gate, (0, 2, 1))
    W1T = jnp.transpose(W1, (0, 1, 3, 2)).reshape(_LAYERS, _EF, _D)
    b1T = b1.reshape(_LAYERS, _EF, 1)
    W2T = jnp.transpose(W2, (0, 3, 1, 2)).reshape(_LAYERS, _D, _EF)
    b2T = jnp.transpose(b2, (0, 2, 1))
    startWT = start_W.reshape(_D, 1)
    startbT = start_b.reshape(_D, 1)
    projWT = jnp.transpose(proj_W, (1, 0))
    projbT = proj_b.reshape(_P, 1)
    xp = x_enc.reshape(_B // _BLK, 1, _T)

    full = lambda shape: pl.BlockSpec(shape, lambda b: (0,) * len(shape))
    dec, bal = pl.pallas_call(
        _model_kernel,
        grid=(_B // _BLK,),
        in_specs=[
            pl.BlockSpec((1, 1, _T), lambda b: (b, 0, 0)),
            full((_D, 1)),
            full((_D, 1)),
            full((_LAYERS, _E, _D)),
            full((_LAYERS, _EF, _D)),
            full((_LAYERS, _EF, 1)),
            full((_LAYERS, _D, _EF)),
            full((_LAYERS, _D, _E)),
            full((_P, _L * _D)),
            full((_P, 1)),
        ],
        out_specs=[
            pl.BlockSpec((_BLK, _P, _N), lambda b: (b, 0, 0)),
            pl.BlockSpec((1, 1), lambda b: (0, 0)),
        ],
        out_shape=[
            jax.ShapeDtypeStruct((_B, _P, _N), f32),
            jax.ShapeDtypeStruct((1, 1), f32),
        ],
        scratch_shapes=[
            pltpu.VMEM((_LAYERS, _E, 1), f32),
            pltpu.VMEM((_LAYERS, _E, 1), f32),
            pltpu.VMEM((_BLK, _L * _D, _N), f32),
        ],
        compiler_params=pltpu.CompilerParams(
            dimension_semantics=("arbitrary",),
        ),
    )(xp, startWT, startbT, wgT, W1T, b1T, W2T, b2T, projWT, projbT)
    return dec, bal[0, 0]


# speedup vs baseline: 1.1017x; 1.0132x over previous
import jax
import jax.numpy as jnp
from jax.experimental import pallas as pl
from jax.experimental.pallas import tpu as pltpu

_B, _L, _N, _D, _F, _E, _K, _P, _LAYERS = 32, 96, 32, 16, 64, 4, 2, 96, 2
_EF = _E * _F
_BLK = 4
_T = _BLK * _L * _N
_TB = _L * _N
_CH = _T


def _top2(lgT):
    f32 = jnp.float32

    def first_max_onehot(x):
        v = jnp.max(x, axis=0, keepdims=True)
        rows = []
        seen = jnp.zeros_like(v)
        for e in range(_E):
            eq = (x[e:e + 1] == v).astype(f32)
            rows.append(eq * (1.0 - seen))
            seen = jnp.maximum(seen, eq)
        return jnp.concatenate(rows, axis=0), v

    oh1, v1 = first_max_onehot(lgT)
    masked = jnp.where(oh1 > 0, -jnp.inf, lgT)
    oh2, v2 = first_max_onehot(masked)
    e2 = jnp.exp(v2 - v1)
    den = 1.0 + e2
    return oh1 * (1.0 / den) + oh2 * (e2 / den)


def _model_kernel(x_ref, startW_ref, startb_ref, wg_ref, W1_ref, b1_ref,
                  W2_ref, b2_ref, projW_ref, projb_ref,
                  dec_ref, bal_ref, imp_ref, load_ref, M_ref):
    b = pl.program_id(0)
    f32 = jnp.float32

    xt = x_ref[0]

    imp_parts = [[] for _ in range(_LAYERS)]
    load_parts = [[] for _ in range(_LAYERS)]
    for c in range(_T // _CH):
        outT = (startW_ref[:] * xt[:, c * _CH:(c + 1) * _CH]
                + startb_ref[:])
        for l in range(_LAYERS):
            lgT = jnp.dot(wg_ref[l], outT, preferred_element_type=f32)
            gatesT = _top2(lgT)
            imp_parts[l].append(jnp.sum(gatesT, axis=1, keepdims=True))
            load_parts[l].append(
                jnp.sum((gatesT > 0).astype(f32), axis=1, keepdims=True))

            hT = jax.nn.gelu(
                jnp.dot(W1_ref[l], outT, preferred_element_type=f32)
                + b1_ref[l])
            yT = None
            for e in range(_E):
                oeT = (jnp.dot(W2_ref[l][:, e * _F:(e + 1) * _F],
                               hT[e * _F:(e + 1) * _F],
                               preferred_element_type=f32)
                       + b2_ref[l][:, e:e + 1])
                term = gatesT[e:e + 1] * oeT
                yT = term if yT is None else yT + term
            outT = outT + yT

        for j in range(_CH // _N):
            off = c * _CH + j * _N
            k, r = off // _TB, (off % _TB) // _N
            M_ref[k, r * _D:(r + 1) * _D, :] = outT[:, j * _N:(j + 1) * _N]
    imps = [sum(p[1:], p[0]) for p in imp_parts]
    loads = [sum(p[1:], p[0]) for p in load_parts]
    for k in range(_BLK):
        dec_ref[k] = (jnp.dot(projW_ref[:], M_ref[k], preferred_element_type=f32)
                      + projb_ref[:])

    @pl.when(b == 0)
    def _init():
        for l in range(_LAYERS):
            imp_ref[l] = imps[l]
            load_ref[l] = loads[l]

    @pl.when(b != 0)
    def _acc():
        for l in range(_LAYERS):
            imp_ref[l] += imps[l]
            load_ref[l] += loads[l]

    @pl.when(b == _B // _BLK - 1)
    def _finalize():
        bal = jnp.zeros((1, 1), dtype=f32)
        for l in range(_LAYERS):
            for ref in (imp_ref, load_ref):
                v = ref[l]
                m = jnp.sum(v, keepdims=True) / _E
                var = jnp.sum((v - m) ** 2, keepdims=True) / (_E - 1)
                bal = bal + var / (m * m + 1e-10)
        bal_ref[:] = bal


def kernel(x_enc, x_mark_enc, x_dec, x_mark_dec, start_W, start_b, w_gate,
           W1, b1, W2, b2, proj_W, proj_b):
    f32 = jnp.float32
    wgT = jnp.transpose(w_gate, (0, 2, 1))
    W1T = jnp.transpose(W1, (0, 1, 3, 2)).reshape(_LAYERS, _EF, _D)
    b1T = b1.reshape(_LAYERS, _EF, 1)
    W2T = jnp.transpose(W2, (0, 3, 1, 2)).reshape(_LAYERS, _D, _EF)
    b2T = jnp.transpose(b2, (0, 2, 1))
    startWT = start_W.reshape(_D, 1)
    startbT = start_b.reshape(_D, 1)
    projWT = jnp.transpose(proj_W, (1, 0))
    projbT = proj_b.reshape(_P, 1)
    xp = x_enc.reshape(_B // _BLK, 1, _T)

    full = lambda shape: pl.BlockSpec(shape, lambda b: (0,) * len(shape))
    dec, bal = pl.pallas_call(
        _model_kernel,
        grid=(_B // _BLK,),
        in_specs=[
            pl.BlockSpec((1, 1, _T), lambda b: (b, 0, 0)),
            full((_D, 1)),
            full((_D, 1)),
            full((_LAYERS, _E, _D)),
            full((_LAYERS, _EF, _D)),
            full((_LAYERS, _EF, 1)),
            full((_LAYERS, _D, _EF)),
            full((_LAYERS, _D, _E)),
            full((_P, _L * _D)),
            full((_P, 1)),
        ],
        out_specs=[
            pl.BlockSpec((_BLK, _P, _N), lambda b: (b, 0, 0)),
            pl.BlockSpec((1, 1), lambda b: (0, 0)),
        ],
        out_shape=[
            jax.ShapeDtypeStruct((_B, _P, _N), f32),
            jax.ShapeDtypeStruct((1, 1), f32),
        ],
        scratch_shapes=[
            pltpu.VMEM((_LAYERS, _E, 1), f32),
            pltpu.VMEM((_LAYERS, _E, 1), f32),
            pltpu.VMEM((_BLK, _L * _D, _N), f32),
        ],
        compiler_params=pltpu.CompilerParams(
            dimension_semantics=("arbitrary",),
        ),
    )(xp, startWT, startbT, wgT, W1T, b1T, W2T, b2T, projWT, projbT)
    return dec, bal[0, 0]


# speedup vs baseline: 1.1240x; 1.0203x over previous
import jax
import jax.numpy as jnp
from jax.experimental import pallas as pl
from jax.experimental.pallas import tpu as pltpu

_B, _L, _N, _D, _F, _E, _K, _P, _LAYERS = 32, 96, 32, 16, 64, 4, 2, 96, 2
_EF = _E * _F
_BLK = 8
_T = _BLK * _L * _N
_TB = _L * _N
_CH = _T


def _top2(lgT):
    f32 = jnp.float32

    def first_max_onehot(x):
        v = jnp.max(x, axis=0, keepdims=True)
        rows = []
        seen = jnp.zeros_like(v)
        for e in range(_E):
            eq = (x[e:e + 1] == v).astype(f32)
            rows.append(eq * (1.0 - seen))
            seen = jnp.maximum(seen, eq)
        return jnp.concatenate(rows, axis=0), v

    oh1, v1 = first_max_onehot(lgT)
    masked = jnp.where(oh1 > 0, -jnp.inf, lgT)
    oh2, v2 = first_max_onehot(masked)
    e2 = jnp.exp(v2 - v1)
    den = 1.0 + e2
    return oh1 * (1.0 / den) + oh2 * (e2 / den)


def _model_kernel(x_ref, startW_ref, startb_ref, wg_ref, W1_ref, b1_ref,
                  W2_ref, b2_ref, projW_ref, projb_ref,
                  dec_ref, bal_ref, imp_ref, load_ref, M_ref):
    b = pl.program_id(0)
    f32 = jnp.float32

    xt = x_ref[0]

    imp_parts = [[] for _ in range(_LAYERS)]
    load_parts = [[] for _ in range(_LAYERS)]
    for c in range(_T // _CH):
        outT = (startW_ref[:] * xt[:, c * _CH:(c + 1) * _CH]
                + startb_ref[:])
        for l in range(_LAYERS):
            lgT = jnp.dot(wg_ref[l], outT, preferred_element_type=f32)
            gatesT = _top2(lgT)
            imp_parts[l].append(jnp.sum(gatesT, axis=1, keepdims=True))
            load_parts[l].append(
                jnp.sum((gatesT > 0).astype(f32), axis=1, keepdims=True))

            hT = jax.nn.gelu(
                jnp.dot(W1_ref[l], outT, preferred_element_type=f32)
                + b1_ref[l])
            yT = None
            for e in range(_E):
                oeT = (jnp.dot(W2_ref[l][:, e * _F:(e + 1) * _F],
                               hT[e * _F:(e + 1) * _F],
                               preferred_element_type=f32)
                       + b2_ref[l][:, e:e + 1])
                term = gatesT[e:e + 1] * oeT
                yT = term if yT is None else yT + term
            outT = outT + yT

        for j in range(_CH // _N):
            off = c * _CH + j * _N
            k, r = off // _TB, (off % _TB) // _N
            M_ref[k, r * _D:(r + 1) * _D, :] = outT[:, j * _N:(j + 1) * _N]
    imps = [sum(p[1:], p[0]) for p in imp_parts]
    loads = [sum(p[1:], p[0]) for p in load_parts]
    for k in range(_BLK):
        dec_ref[k] = (jnp.dot(projW_ref[:], M_ref[k], preferred_element_type=f32)
                      + projb_ref[:])

    @pl.when(b == 0)
    def _init():
        for l in range(_LAYERS):
            imp_ref[l] = imps[l]
            load_ref[l] = loads[l]

    @pl.when(b != 0)
    def _acc():
        for l in range(_LAYERS):
            imp_ref[l] += imps[l]
            load_ref[l] += loads[l]

    @pl.when(b == _B // _BLK - 1)
    def _finalize():
        bal = jnp.zeros((1, 1), dtype=f32)
        for l in range(_LAYERS):
            for ref in (imp_ref, load_ref):
                v = ref[l]
                m = jnp.sum(v, keepdims=True) / _E
                var = jnp.sum((v - m) ** 2, keepdims=True) / (_E - 1)
                bal = bal + var / (m * m + 1e-10)
        bal_ref[:] = bal


def kernel(x_enc, x_mark_enc, x_dec, x_mark_dec, start_W, start_b, w_gate,
           W1, b1, W2, b2, proj_W, proj_b):
    f32 = jnp.float32
    wgT = jnp.transpose(w_gate, (0, 2, 1))
    W1T = jnp.transpose(W1, (0, 1, 3, 2)).reshape(_LAYERS, _EF, _D)
    b1T = b1.reshape(_LAYERS, _EF, 1)
    W2T = jnp.transpose(W2, (0, 3, 1, 2)).reshape(_LAYERS, _D, _EF)
    b2T = jnp.transpose(b2, (0, 2, 1))
    startWT = start_W.reshape(_D, 1)
    startbT = start_b.reshape(_D, 1)
    projWT = jnp.transpose(proj_W, (1, 0))
    projbT = proj_b.reshape(_P, 1)
    xp = x_enc.reshape(_B // _BLK, 1, _T)

    full = lambda shape: pl.BlockSpec(shape, lambda b: (0,) * len(shape))
    dec, bal = pl.pallas_call(
        _model_kernel,
        grid=(_B // _BLK,),
        in_specs=[
            pl.BlockSpec((1, 1, _T), lambda b: (b, 0, 0)),
            full((_D, 1)),
            full((_D, 1)),
            full((_LAYERS, _E, _D)),
            full((_LAYERS, _EF, _D)),
            full((_LAYERS, _EF, 1)),
            full((_LAYERS, _D, _EF)),
            full((_LAYERS, _D, _E)),
            full((_P, _L * _D)),
            full((_P, 1)),
        ],
        out_specs=[
            pl.BlockSpec((_BLK, _P, _N), lambda b: (b, 0, 0)),
            pl.BlockSpec((1, 1), lambda b: (0, 0)),
        ],
        out_shape=[
            jax.ShapeDtypeStruct((_B, _P, _N), f32),
            jax.ShapeDtypeStruct((1, 1), f32),
        ],
        scratch_shapes=[
            pltpu.VMEM((_LAYERS, _E, 1), f32),
            pltpu.VMEM((_LAYERS, _E, 1), f32),
            pltpu.VMEM((_BLK, _L * _D, _N), f32),
        ],
        compiler_params=pltpu.CompilerParams(
            dimension_semantics=("arbitrary",),
        ),
    )(xp, startWT, startbT, wgT, W1T, b1T, W2T, b2T, projWT, projbT)
    return dec, bal[0, 0]
